# baseline (speedup 1.0000x reference)
import functools

import jax
import jax.numpy as jnp
from jax import lax
from jax.experimental import pallas as pl
from jax.experimental.pallas import tpu as pltpu
from jax.experimental.pallas import tpu_sc as plsc

_N = 20000
_NW = 16
_PER_W = 1280
_NPAD = _NW * _PER_W
_NV = _PER_W // 16
_MAX_OUT = 100
_IOU_THRESHOLD = 0.5
_BIG = 1 << 30


def _sc_nms(x1_hbm, y1_hbm, x2_hbm, y2_hbm, s_hbm, out_hbm,
            x1v, y1v, x2v, y2v, av, msv, candv, pubv, rowacc, shared):
    core = lax.axis_index("c")
    wid = lax.axis_index("s")
    base = wid * _PER_W
    lane = lax.iota(jnp.int32, 16)

    pltpu.sync_copy(x1_hbm.at[pl.ds(base, _PER_W)], x1v)
    pltpu.sync_copy(y1_hbm.at[pl.ds(base, _PER_W)], y1v)
    pltpu.sync_copy(x2_hbm.at[pl.ds(base, _PER_W)], x2v)
    pltpu.sync_copy(y2_hbm.at[pl.ds(base, _PER_W)], y2v)
    pltpu.sync_copy(s_hbm.at[pl.ds(base, _PER_W)], msv)

    mx0 = jnp.full((16,), -3.0, jnp.float32)
    ix0 = jnp.zeros((16,), jnp.int32)

    @plsc.parallel_loop(0, _NV, 1, unroll=16, carry=(mx0, ix0))
    def init_j(j, carry):
        mx, ix = carry
        sl = pl.ds(j * 16, 16)
        a = (jnp.maximum(x2v[sl] - x1v[sl], 0.0)
             * jnp.maximum(y2v[sl] - y1v[sl], 0.0))
        av[sl] = a
        ms = msv[sl]
        gt = ms > mx
        mx = jnp.where(gt, ms, mx)
        ix = jnp.where(gt, j, ix)
        return mx, ix

    mx, ix = init_j

    def pick(i, carry):
        mx, ix = carry
        mloc = jnp.max(mx)
        lflat = ix * 16 + lane
        lidx = jnp.min(jnp.where(mx == mloc, lflat, _BIG))
        iv = jnp.full((16,), lidx, jnp.int32)
        cx1 = plsc.load_gather(x1v, [iv])
        cy1 = plsc.load_gather(y1v, [iv])
        cx2 = plsc.load_gather(x2v, [iv])
        cy2 = plsc.load_gather(y2v, [iv])
        ca = plsc.load_gather(av, [iv])
        mlv = jnp.full((16,), mloc, jnp.float32)
        pub = jnp.where(lane == 0, mlv,
              jnp.where(lane == 1, cx1,
              jnp.where(lane == 2, cy1,
              jnp.where(lane == 3, cx2,
              jnp.where(lane == 4, cy2,
              jnp.where(lane == 5, ca, 0.0))))))
        pubv[...] = pub
        par = lax.rem(i, 2)
        pltpu.sync_copy(pubv, shared.at[pl.ds(par * 256 + wid * 16, 16)])
        plsc.subcore_barrier()
        pltpu.sync_copy(shared.at[pl.ds(par * 256, 256)], candv)

        zeros16 = jnp.zeros((16,), jnp.int32)
        svec = plsc.load_gather(candv, [lane * 16])
        mg = jnp.max(svec)
        wstar = jnp.min(jnp.where(svec == mg, lane, _BIG))
        wbase = wstar * 16
        gx1 = plsc.load_gather(candv, [jnp.full((16,), wbase + 1, jnp.int32)])
        gy1 = plsc.load_gather(candv, [jnp.full((16,), wbase + 2, jnp.int32)])
        gx2 = plsc.load_gather(candv, [jnp.full((16,), wbase + 3, jnp.int32)])
        gy2 = plsc.load_gather(candv, [jnp.full((16,), wbase + 4, jnp.int32)])
        ga = plsc.load_gather(candv, [jnp.full((16,), wbase + 5, jnp.int32)])
        has = mg > -1.0
        hasmask = jnp.full((16,), has)

        @pl.when(jnp.logical_and(core == 0, wid == 0))
        def _():
            hf = jnp.where(hasmask, 1.0, 0.0)
            shift = jnp.where(lane < 4, lane + 1,
                              jnp.where(lane == 4, 0, 6))
            row = plsc.load_gather(candv, [wbase + shift])
            rowacc[pl.ds(i * 16, 16)] = jnp.where(lane < 5, row * hf, 0.0)

        @plsc.parallel_loop(0, _NV, 1, unroll=16, carry=(mx0, ix0))
        def upd(j, carry2):
            mx2, ix2 = carry2
            sl = pl.ds(j * 16, 16)
            x1 = x1v[sl]
            y1 = y1v[sl]
            x2 = x2v[sl]
            y2 = y2v[sl]
            a = av[sl]
            ms = msv[sl]
            xx1 = jnp.maximum(gx1, x1)
            yy1 = jnp.maximum(gy1, y1)
            xx2 = jnp.minimum(gx2, x2)
            yy2 = jnp.minimum(gy2, y2)
            inter = (jnp.maximum(xx2 - xx1, 0.0)
                     * jnp.maximum(yy2 - yy1, 0.0))
            iou = inter / (ga + a - inter + 1e-9)
            kill = jnp.logical_and(iou > _IOU_THRESHOLD, hasmask)
            ms = jnp.where(kill, -2.0, ms)
            msv[sl] = ms
            gt = ms > mx2
            mx2 = jnp.where(gt, ms, mx2)
            ix2 = jnp.where(gt, j, ix2)
            return mx2, ix2

        return upd

    lax.fori_loop(0, _MAX_OUT, pick, (mx, ix))

    @pl.when(jnp.logical_and(core == 0, wid == 0))
    def _():
        pltpu.sync_copy(rowacc, out_hbm)


_sc_call = functools.partial(
    pl.kernel,
    out_type=jax.ShapeDtypeStruct((_MAX_OUT * 16,), jnp.float32),
    mesh=plsc.VectorSubcoreMesh(core_axis_name="c", subcore_axis_name="s",
                                num_cores=2, num_subcores=16),
    scratch_types=[
        pltpu.VMEM((_PER_W,), jnp.float32),
        pltpu.VMEM((_PER_W,), jnp.float32),
        pltpu.VMEM((_PER_W,), jnp.float32),
        pltpu.VMEM((_PER_W,), jnp.float32),
        pltpu.VMEM((_PER_W,), jnp.float32),
        pltpu.VMEM((_PER_W,), jnp.float32),
        pltpu.VMEM((_NW * 16,), jnp.float32),
        pltpu.VMEM((16,), jnp.float32),
        pltpu.VMEM((_MAX_OUT * 16,), jnp.float32),
        pltpu.VMEM_SHARED((2 * _NW * 16,), jnp.float32),
    ],
    compiler_params=pltpu.CompilerParams(needs_layout_passes=False),
)


def kernel(boxes, scores):
    bp = jnp.pad(boxes, ((0, _NPAD - _N), (0, 0)))
    sp = jnp.pad(scores, (0, _NPAD - _N), constant_values=-2.0)
    planes = [bp[:, k] for k in range(4)]
    out = _sc_call(_sc_nms)(planes[0], planes[1], planes[2], planes[3], sp)
    return out.reshape(_MAX_OUT, 16)[:, :5]


# speedup vs baseline: 1.0201x; 1.0201x over previous
import functools

import jax
import jax.numpy as jnp
from jax import lax
from jax.experimental import pallas as pl
from jax.experimental.pallas import tpu as pltpu
from jax.experimental.pallas import tpu_sc as plsc

_N = 20000
_NW = 16
_PER_W = 1280
_NPAD = _NW * _PER_W
_NV = _PER_W // 16
_MAX_OUT = 100
_IOU_THRESHOLD = 0.5
_BIG = 1 << 30


def _sc_nms(x1_hbm, y1_hbm, x2_hbm, y2_hbm, s_hbm, out_hbm,
            x1v, y1v, x2v, y2v, av, msv, candv, pubv, rowacc, shared):
    core = lax.axis_index("c")
    wid = lax.axis_index("s")
    base = wid * _PER_W
    lane = lax.iota(jnp.int32, 16)

    pltpu.sync_copy(x1_hbm.at[pl.ds(base, _PER_W)], x1v)
    pltpu.sync_copy(y1_hbm.at[pl.ds(base, _PER_W)], y1v)
    pltpu.sync_copy(x2_hbm.at[pl.ds(base, _PER_W)], x2v)
    pltpu.sync_copy(y2_hbm.at[pl.ds(base, _PER_W)], y2v)
    pltpu.sync_copy(s_hbm.at[pl.ds(base, _PER_W)], msv)

    mx0 = jnp.full((16,), -3.0, jnp.float32)
    ix0 = jnp.zeros((16,), jnp.int32)

    @plsc.parallel_loop(0, _NV, 1, unroll=8, carry=(mx0, ix0))
    def init_j(j, carry):
        mx, ix = carry
        sl = pl.ds(j * 16, 16)
        a = (jnp.maximum(x2v[sl] - x1v[sl], 0.0)
             * jnp.maximum(y2v[sl] - y1v[sl], 0.0))
        av[sl] = a
        ms = msv[sl]
        gt = ms > mx
        mx = jnp.where(gt, ms, mx)
        ix = jnp.where(gt, j, ix)
        return mx, ix

    mx, ix = init_j

    def pick(i, carry):
        mx, ix = carry
        mloc = jnp.max(mx)
        lflat = ix * 16 + lane
        lidx = jnp.min(jnp.where(mx == mloc, lflat, _BIG))
        iv = jnp.full((16,), lidx, jnp.int32)
        cx1 = plsc.load_gather(x1v, [iv])
        cy1 = plsc.load_gather(y1v, [iv])
        cx2 = plsc.load_gather(x2v, [iv])
        cy2 = plsc.load_gather(y2v, [iv])
        ca = plsc.load_gather(av, [iv])
        mlv = jnp.full((16,), mloc, jnp.float32)
        pub = jnp.where(lane == 0, mlv,
              jnp.where(lane == 1, cx1,
              jnp.where(lane == 2, cy1,
              jnp.where(lane == 3, cx2,
              jnp.where(lane == 4, cy2,
              jnp.where(lane == 5, ca, 0.0))))))
        pubv[...] = pub
        par = lax.rem(i, 2)
        pltpu.sync_copy(pubv, shared.at[pl.ds(par * 256 + wid * 16, 16)])
        plsc.subcore_barrier()
        pltpu.sync_copy(shared.at[pl.ds(par * 256, 256)], candv)

        zeros16 = jnp.zeros((16,), jnp.int32)
        svec = plsc.load_gather(candv, [lane * 16])
        mg = jnp.max(svec)
        wstar = jnp.min(jnp.where(svec == mg, lane, _BIG))
        wbase = wstar * 16
        gx1 = plsc.load_gather(candv, [jnp.full((16,), wbase + 1, jnp.int32)])
        gy1 = plsc.load_gather(candv, [jnp.full((16,), wbase + 2, jnp.int32)])
        gx2 = plsc.load_gather(candv, [jnp.full((16,), wbase + 3, jnp.int32)])
        gy2 = plsc.load_gather(candv, [jnp.full((16,), wbase + 4, jnp.int32)])
        ga = plsc.load_gather(candv, [jnp.full((16,), wbase + 5, jnp.int32)])
        has = mg > -1.0
        hasmask = jnp.full((16,), has)

        @pl.when(jnp.logical_and(core == 0, wid == 0))
        def _():
            hf = jnp.where(hasmask, 1.0, 0.0)
            shift = jnp.where(lane < 4, lane + 1,
                              jnp.where(lane == 4, 0, 6))
            row = plsc.load_gather(candv, [wbase + shift])
            rowacc[pl.ds(i * 16, 16)] = jnp.where(lane < 5, row * hf, 0.0)

        @plsc.parallel_loop(0, _NV, 1, unroll=8, carry=(mx0, ix0))
        def upd(j, carry2):
            mx2, ix2 = carry2
            sl = pl.ds(j * 16, 16)
            x1 = x1v[sl]
            y1 = y1v[sl]
            x2 = x2v[sl]
            y2 = y2v[sl]
            a = av[sl]
            ms = msv[sl]
            xx1 = jnp.maximum(gx1, x1)
            yy1 = jnp.maximum(gy1, y1)
            xx2 = jnp.minimum(gx2, x2)
            yy2 = jnp.minimum(gy2, y2)
            inter = (jnp.maximum(xx2 - xx1, 0.0)
                     * jnp.maximum(yy2 - yy1, 0.0))
            iou = inter / (ga + a - inter + 1e-9)
            kill = jnp.logical_and(iou > _IOU_THRESHOLD, hasmask)
            ms = jnp.where(kill, -2.0, ms)
            msv[sl] = ms
            gt = ms > mx2
            mx2 = jnp.where(gt, ms, mx2)
            ix2 = jnp.where(gt, j, ix2)
            return mx2, ix2

        return upd

    lax.fori_loop(0, _MAX_OUT, pick, (mx, ix))

    @pl.when(jnp.logical_and(core == 0, wid == 0))
    def _():
        pltpu.sync_copy(rowacc, out_hbm)


_sc_call = functools.partial(
    pl.kernel,
    out_type=jax.ShapeDtypeStruct((_MAX_OUT * 16,), jnp.float32),
    mesh=plsc.VectorSubcoreMesh(core_axis_name="c", subcore_axis_name="s",
                                num_cores=2, num_subcores=16),
    scratch_types=[
        pltpu.VMEM((_PER_W,), jnp.float32),
        pltpu.VMEM((_PER_W,), jnp.float32),
        pltpu.VMEM((_PER_W,), jnp.float32),
        pltpu.VMEM((_PER_W,), jnp.float32),
        pltpu.VMEM((_PER_W,), jnp.float32),
        pltpu.VMEM((_PER_W,), jnp.float32),
        pltpu.VMEM((_NW * 16,), jnp.float32),
        pltpu.VMEM((16,), jnp.float32),
        pltpu.VMEM((_MAX_OUT * 16,), jnp.float32),
        pltpu.VMEM_SHARED((2 * _NW * 16,), jnp.float32),
    ],
    compiler_params=pltpu.CompilerParams(needs_layout_passes=False),
)


def kernel(boxes, scores):
    bp = jnp.pad(boxes, ((0, _NPAD - _N), (0, 0)))
    sp = jnp.pad(scores, (0, _NPAD - _N), constant_values=-2.0)
    planes = [bp[:, k] for k in range(4)]
    out = _sc_call(_sc_nms)(planes[0], planes[1], planes[2], planes[3], sp)
    return out.reshape(_MAX_OUT, 16)[:, :5]


# speedup vs baseline: 1.0273x; 1.0070x over previous
import functools

import jax
import jax.numpy as jnp
from jax import lax
from jax.experimental import pallas as pl
from jax.experimental.pallas import tpu as pltpu
from jax.experimental.pallas import tpu_sc as plsc

_N = 20000
_NW = 16
_PER_W = 1280
_NPAD = _NW * _PER_W
_NV = _PER_W // 16
_MAX_OUT = 100
_IOU_THRESHOLD = 0.5
_BIG = 1 << 30


def _sc_nms(x1_hbm, y1_hbm, x2_hbm, y2_hbm, s_hbm, out_hbm,
            x1v, y1v, x2v, y2v, av, msv, candv, pubv, rowacc, shared):
    core = lax.axis_index("c")
    wid = lax.axis_index("s")
    base = wid * _PER_W
    lane = lax.iota(jnp.int32, 16)

    pltpu.sync_copy(x1_hbm.at[pl.ds(base, _PER_W)], x1v)
    pltpu.sync_copy(y1_hbm.at[pl.ds(base, _PER_W)], y1v)
    pltpu.sync_copy(x2_hbm.at[pl.ds(base, _PER_W)], x2v)
    pltpu.sync_copy(y2_hbm.at[pl.ds(base, _PER_W)], y2v)
    pltpu.sync_copy(s_hbm.at[pl.ds(base, _PER_W)], msv)

    mx0 = jnp.full((16,), -3.0, jnp.float32)
    ix0 = jnp.zeros((16,), jnp.int32)

    @plsc.parallel_loop(0, _NV, 1, unroll=8, carry=(mx0, ix0))
    def init_j(j, carry):
        mx, ix = carry
        sl = pl.ds(j * 16, 16)
        a = (jnp.maximum(x2v[sl] - x1v[sl], 0.0)
             * jnp.maximum(y2v[sl] - y1v[sl], 0.0))
        av[sl] = a
        ms = msv[sl]
        gt = ms > mx
        mx = jnp.where(gt, ms, mx)
        ix = jnp.where(gt, j, ix)
        return mx, ix

    mx, ix = init_j

    def pick(i, carry):
        mx, ix = carry
        mloc = jnp.max(mx)
        lflat = ix * 16 + lane
        lidx = jnp.min(jnp.where(mx == mloc, lflat, _BIG))
        iv = jnp.full((16,), lidx, jnp.int32)
        cx1 = plsc.load_gather(x1v, [iv])
        cy1 = plsc.load_gather(y1v, [iv])
        cx2 = plsc.load_gather(x2v, [iv])
        cy2 = plsc.load_gather(y2v, [iv])
        ca = plsc.load_gather(av, [iv])
        mlv = jnp.full((16,), mloc, jnp.float32)
        pub = jnp.where(lane == 0, mlv,
              jnp.where(lane == 1, cx1,
              jnp.where(lane == 2, cy1,
              jnp.where(lane == 3, cx2,
              jnp.where(lane == 4, cy2,
              jnp.where(lane == 5, ca, 0.0))))))
        pubv[...] = pub
        par = lax.rem(i, 2)
        pltpu.sync_copy(pubv, shared.at[pl.ds(par * 256 + wid * 16, 16)])
        plsc.subcore_barrier()
        pltpu.sync_copy(shared.at[pl.ds(par * 256, 256)], candv)

        zeros16 = jnp.zeros((16,), jnp.int32)
        svec = plsc.load_gather(candv, [lane * 16])
        mg = jnp.max(svec)
        wstar = jnp.min(jnp.where(svec == mg, lane, _BIG))
        wbase = wstar * 16
        gx1 = plsc.load_gather(candv, [jnp.full((16,), wbase + 1, jnp.int32)])
        gy1 = plsc.load_gather(candv, [jnp.full((16,), wbase + 2, jnp.int32)])
        gx2 = plsc.load_gather(candv, [jnp.full((16,), wbase + 3, jnp.int32)])
        gy2 = plsc.load_gather(candv, [jnp.full((16,), wbase + 4, jnp.int32)])
        ga = plsc.load_gather(candv, [jnp.full((16,), wbase + 5, jnp.int32)])
        has = mg > -1.0
        hasmask = jnp.full((16,), has)

        @pl.when(jnp.logical_and(core == 0, wid == 0))
        def _():
            hf = jnp.where(hasmask, 1.0, 0.0)
            shift = jnp.where(lane < 4, lane + 1,
                              jnp.where(lane == 4, 0, 6))
            row = plsc.load_gather(candv, [wbase + shift])
            rowacc[pl.ds(i * 16, 16)] = jnp.where(lane < 5, row * hf, 0.0)

        @plsc.parallel_loop(0, _NV, 1, unroll=4, carry=(mx0, ix0))
        def upd(j, carry2):
            mx2, ix2 = carry2
            sl = pl.ds(j * 16, 16)
            x1 = x1v[sl]
            y1 = y1v[sl]
            x2 = x2v[sl]
            y2 = y2v[sl]
            a = av[sl]
            ms = msv[sl]
            xx1 = jnp.maximum(gx1, x1)
            yy1 = jnp.maximum(gy1, y1)
            xx2 = jnp.minimum(gx2, x2)
            yy2 = jnp.minimum(gy2, y2)
            inter = (jnp.maximum(xx2 - xx1, 0.0)
                     * jnp.maximum(yy2 - yy1, 0.0))
            iou = inter / (ga + a - inter + 1e-9)
            kill = jnp.logical_and(iou > _IOU_THRESHOLD, hasmask)
            ms = jnp.where(kill, -2.0, ms)
            msv[sl] = ms
            gt = ms > mx2
            mx2 = jnp.where(gt, ms, mx2)
            ix2 = jnp.where(gt, j, ix2)
            return mx2, ix2

        return upd

    lax.fori_loop(0, _MAX_OUT, pick, (mx, ix))

    @pl.when(jnp.logical_and(core == 0, wid == 0))
    def _():
        pltpu.sync_copy(rowacc, out_hbm)


_sc_call = functools.partial(
    pl.kernel,
    out_type=jax.ShapeDtypeStruct((_MAX_OUT * 16,), jnp.float32),
    mesh=plsc.VectorSubcoreMesh(core_axis_name="c", subcore_axis_name="s",
                                num_cores=2, num_subcores=16),
    scratch_types=[
        pltpu.VMEM((_PER_W,), jnp.float32),
        pltpu.VMEM((_PER_W,), jnp.float32),
        pltpu.VMEM((_PER_W,), jnp.float32),
        pltpu.VMEM((_PER_W,), jnp.float32),
        pltpu.VMEM((_PER_W,), jnp.float32),
        pltpu.VMEM((_PER_W,), jnp.float32),
        pltpu.VMEM((_NW * 16,), jnp.float32),
        pltpu.VMEM((16,), jnp.float32),
        pltpu.VMEM((_MAX_OUT * 16,), jnp.float32),
        pltpu.VMEM_SHARED((2 * _NW * 16,), jnp.float32),
    ],
    compiler_params=pltpu.CompilerParams(needs_layout_passes=False),
)


def kernel(boxes, scores):
    bp = jnp.pad(boxes, ((0, _NPAD - _N), (0, 0)))
    sp = jnp.pad(scores, (0, _NPAD - _N), constant_values=-2.0)
    planes = [bp[:, k] for k in range(4)]
    out = _sc_call(_sc_nms)(planes[0], planes[1], planes[2], planes[3], sp)
    return out.reshape(_MAX_OUT, 16)[:, :5]


# speedup vs baseline: 1.0357x; 1.0082x over previous
import functools

import jax
import jax.numpy as jnp
from jax import lax
from jax.experimental import pallas as pl
from jax.experimental.pallas import tpu as pltpu
from jax.experimental.pallas import tpu_sc as plsc

_N = 20000
_NW = 16
_PER_W = 1280
_NPAD = _NW * _PER_W
_NV = _PER_W // 16
_MAX_OUT = 100
_IOU_THRESHOLD = 0.5
_BIG = 1 << 30


def _sc_nms(x1_hbm, y1_hbm, x2_hbm, y2_hbm, s_hbm, out_hbm,
            x1v, y1v, x2v, y2v, av, msv, candv, pubv, rowacc, shared):
    core = lax.axis_index("c")
    wid = lax.axis_index("s")
    base = wid * _PER_W
    lane = lax.iota(jnp.int32, 16)

    pltpu.sync_copy(x1_hbm.at[pl.ds(base, _PER_W)], x1v)
    pltpu.sync_copy(y1_hbm.at[pl.ds(base, _PER_W)], y1v)
    pltpu.sync_copy(x2_hbm.at[pl.ds(base, _PER_W)], x2v)
    pltpu.sync_copy(y2_hbm.at[pl.ds(base, _PER_W)], y2v)
    pltpu.sync_copy(s_hbm.at[pl.ds(base, _PER_W)], msv)

    mx0 = jnp.full((16,), -3.0, jnp.float32)
    ix0 = jnp.zeros((16,), jnp.int32)

    @plsc.parallel_loop(0, _NV, 1, unroll=8, carry=(mx0, ix0))
    def init_j(j, carry):
        mx, ix = carry
        sl = pl.ds(j * 16, 16)
        a = (jnp.maximum(x2v[sl] - x1v[sl], 0.0)
             * jnp.maximum(y2v[sl] - y1v[sl], 0.0))
        av[sl] = a
        ms = msv[sl]
        gt = ms > mx
        mx = jnp.where(gt, ms, mx)
        ix = jnp.where(gt, j, ix)
        return mx, ix

    mx, ix = init_j

    def pick(i, carry):
        mx, ix = carry
        mloc = jnp.max(mx)
        lflat = ix * 16 + lane
        lidx = jnp.min(jnp.where(mx == mloc, lflat, _BIG))
        iv = jnp.full((16,), lidx, jnp.int32)
        cx1 = plsc.load_gather(x1v, [iv])
        cy1 = plsc.load_gather(y1v, [iv])
        cx2 = plsc.load_gather(x2v, [iv])
        cy2 = plsc.load_gather(y2v, [iv])
        ca = plsc.load_gather(av, [iv])
        mlv = jnp.full((16,), mloc, jnp.float32)
        pub = jnp.where(lane == 0, mlv,
              jnp.where(lane == 1, cx1,
              jnp.where(lane == 2, cy1,
              jnp.where(lane == 3, cx2,
              jnp.where(lane == 4, cy2,
              jnp.where(lane == 5, ca, 0.0))))))
        pubv[...] = pub
        par = lax.rem(i, 2)
        pltpu.sync_copy(pubv, shared.at[pl.ds(par * 256 + wid * 16, 16)])
        plsc.subcore_barrier()
        pltpu.sync_copy(shared.at[pl.ds(par * 256, 256)], candv)

        zeros16 = jnp.zeros((16,), jnp.int32)
        svec = plsc.load_gather(candv, [lane * 16])
        mg = jnp.max(svec)
        wstar = jnp.min(jnp.where(svec == mg, lane, _BIG))
        wbase = wstar * 16
        gx1 = plsc.load_gather(candv, [jnp.full((16,), wbase + 1, jnp.int32)])
        gy1 = plsc.load_gather(candv, [jnp.full((16,), wbase + 2, jnp.int32)])
        gx2 = plsc.load_gather(candv, [jnp.full((16,), wbase + 3, jnp.int32)])
        gy2 = plsc.load_gather(candv, [jnp.full((16,), wbase + 4, jnp.int32)])
        ga = plsc.load_gather(candv, [jnp.full((16,), wbase + 5, jnp.int32)])
        has = mg > -1.0
        hasmask = jnp.full((16,), has)

        @pl.when(jnp.logical_and(core == 0, wid == 0))
        def _():
            hf = jnp.where(hasmask, 1.0, 0.0)
            shift = jnp.where(lane < 4, lane + 1,
                              jnp.where(lane == 4, 0, 6))
            row = plsc.load_gather(candv, [wbase + shift])
            rowacc[pl.ds(i * 16, 16)] = jnp.where(lane < 5, row * hf, 0.0)

        @plsc.parallel_loop(0, _NV, 1, unroll=4, carry=(mx0, ix0))
        def upd(j, carry2):
            mx2, ix2 = carry2
            sl = pl.ds(j * 16, 16)
            x1 = x1v[sl]
            y1 = y1v[sl]
            x2 = x2v[sl]
            y2 = y2v[sl]
            a = av[sl]
            ms = msv[sl]
            xx1 = jnp.maximum(gx1, x1)
            yy1 = jnp.maximum(gy1, y1)
            xx2 = jnp.minimum(gx2, x2)
            yy2 = jnp.minimum(gy2, y2)
            inter = (jnp.maximum(xx2 - xx1, 0.0)
                     * jnp.maximum(yy2 - yy1, 0.0))
            iou = inter / (ga + a - inter + 1e-9)
            ms = jnp.where(iou > _IOU_THRESHOLD, -2.0, ms)
            msv[sl] = ms
            gt = ms > mx2
            mx2 = jnp.where(gt, ms, mx2)
            ix2 = jnp.where(gt, j, ix2)
            return mx2, ix2

        return upd

    lax.fori_loop(0, _MAX_OUT, pick, (mx, ix))

    @pl.when(jnp.logical_and(core == 0, wid == 0))
    def _():
        pltpu.sync_copy(rowacc, out_hbm)


_sc_call = functools.partial(
    pl.kernel,
    out_type=jax.ShapeDtypeStruct((_MAX_OUT * 16,), jnp.float32),
    mesh=plsc.VectorSubcoreMesh(core_axis_name="c", subcore_axis_name="s",
                                num_cores=2, num_subcores=16),
    scratch_types=[
        pltpu.VMEM((_PER_W,), jnp.float32),
        pltpu.VMEM((_PER_W,), jnp.float32),
        pltpu.VMEM((_PER_W,), jnp.float32),
        pltpu.VMEM((_PER_W,), jnp.float32),
        pltpu.VMEM((_PER_W,), jnp.float32),
        pltpu.VMEM((_PER_W,), jnp.float32),
        pltpu.VMEM((_NW * 16,), jnp.float32),
        pltpu.VMEM((16,), jnp.float32),
        pltpu.VMEM((_MAX_OUT * 16,), jnp.float32),
        pltpu.VMEM_SHARED((2 * _NW * 16,), jnp.float32),
    ],
    compiler_params=pltpu.CompilerParams(needs_layout_passes=False),
)


def kernel(boxes, scores):
    bp = jnp.pad(boxes, ((0, _NPAD - _N), (0, 0)))
    sp = jnp.pad(scores, (0, _NPAD - _N), constant_values=-2.0)
    planes = [bp[:, k] for k in range(4)]
    out = _sc_call(_sc_nms)(planes[0], planes[1], planes[2], planes[3], sp)
    return out.reshape(_MAX_OUT, 16)[:, :5]
